# baseline (speedup 1.0000x reference)
import functools

import jax
import jax.numpy as jnp
from jax import lax
from jax.experimental import pallas as pl
from jax.experimental.pallas import tpu as pltpu
from jax.experimental.pallas import tpu_sc as plsc

N = 10000
E = 320000
D_IN = 128
D_HID = 256
D_OUT = 128

NUM_SUBCORES = 16
EDGES_PER_SUB = E // NUM_SUBCORES
CHUNK = 128
IDX_BLOCK = 16
NUM_BLOCKS = 10
CHUNKS_PER_SUB = IDX_BLOCK * NUM_BLOCKS
EPS = CHUNK * CHUNKS_PER_SUB
PAD_E = EPS - EDGES_PER_SUB
NPAD = 10240
ROWS_PER_SUB = NPAD // NUM_SUBCORES


def _seg_sum_dual(data, gi, si, zslab):
    mesh = plsc.VectorSubcoreMesh(core_axis_name="c", subcore_axis_name="s")

    @functools.partial(
        pl.kernel,
        mesh=mesh,
        out_type=jax.ShapeDtypeStruct((2, NPAD, 128), jnp.float32),
        scratch_types=[
            pltpu.VMEM((2, IDX_BLOCK, CHUNK), jnp.int32),
            pltpu.VMEM((2, IDX_BLOCK, CHUNK), jnp.int32),
            pltpu.VMEM((CHUNK, 128), jnp.float32),
            pltpu.VMEM((CHUNK, 128), jnp.float32),
            pltpu.VMEM_SHARED((NPAD, 128), jnp.float32),
            pltpu.SemaphoreType.DMA,
            pltpu.SemaphoreType.DMA,
            pltpu.SemaphoreType.DMA,
            pltpu.SemaphoreType.DMA,
        ],
    )
    def seg_kernel(data_hbm, gi_hbm, si_hbm, z_hbm, out_hbm,
                   gi_v, si_v, rows0, rows1, acc_sh,
                   gsem0, gsem1, isem_g, isem_s):
        c = lax.axis_index("c")
        s = lax.axis_index("s")
        w = c * NUM_SUBCORES + s

        def i_start(b, p):
            row0 = w * CHUNKS_PER_SUB + b * IDX_BLOCK
            pltpu.async_copy(gi_hbm.at[pl.ds(row0, IDX_BLOCK)], gi_v.at[p], isem_g)
            pltpu.async_copy(si_hbm.at[pl.ds(row0, IDX_BLOCK)], si_v.at[p], isem_s)

        def i_wait(p):
            pltpu.make_async_copy(gi_hbm.at[pl.ds(0, IDX_BLOCK)], gi_v.at[p], isem_g).wait()
            pltpu.make_async_copy(si_hbm.at[pl.ds(0, IDX_BLOCK)], si_v.at[p], isem_s).wait()

        def g_start(idx_row, buf, sem):
            pltpu.async_copy(data_hbm.at[idx_row], buf, sem)

        def g_wait(buf, sem):
            pltpu.make_async_copy(data_hbm.at[gi_v.at[0, 0]], buf, sem).wait()

        i_start(0, 0)
        pltpu.sync_copy(z_hbm, acc_sh.at[pl.ds(s * ROWS_PER_SUB, ROWS_PER_SUB)])
        plsc.subcore_barrier()
        i_wait(0)
        g_start(gi_v.at[0, 0], rows0, gsem0)
        g_start(gi_v.at[0, 1], rows1, gsem1)

        @pl.loop(0, NUM_BLOCKS)
        def _(b):
            p = lax.rem(b, 2)

            @pl.when(b + 1 < NUM_BLOCKS)
            def _():
                i_start(b + 1, 1 - p)

            @pl.loop(0, IDX_BLOCK, step=2)
            def _(k):
                bridge = jnp.logical_and(k + 2 >= IDX_BLOCK, b + 1 < NUM_BLOCKS)
                g_wait(rows0, gsem0)
                pltpu.sync_copy(rows0, acc_sh.at[si_v.at[p, k]], add=True)

                @pl.when(k + 2 < IDX_BLOCK)
                def _():
                    g_start(gi_v.at[p, k + 2], rows0, gsem0)

                @pl.when(bridge)
                def _():
                    i_wait(1 - p)
                    g_start(gi_v.at[1 - p, 0], rows0, gsem0)

                g_wait(rows1, gsem1)
                pltpu.sync_copy(rows1, acc_sh.at[si_v.at[p, k + 1]], add=True)

                @pl.when(k + 3 < IDX_BLOCK)
                def _():
                    g_start(gi_v.at[p, k + 3], rows1, gsem1)

                @pl.when(bridge)
                def _():
                    g_start(gi_v.at[1 - p, 1], rows1, gsem1)

        plsc.subcore_barrier()
        pltpu.sync_copy(
            acc_sh.at[pl.ds(s * ROWS_PER_SUB, ROWS_PER_SUB)],
            out_hbm.at[c, pl.ds(s * ROWS_PER_SUB, ROWS_PER_SUB)],
        )

    return seg_kernel(data, gi, si, zslab)


def _build_idx(g0, s0, g1, s1):
    pad_g = (jnp.arange(PAD_E, dtype=jnp.int32) % 1024)
    pad_s = (N + jnp.arange(PAD_E, dtype=jnp.int32) % (NPAD - N))

    def planes(pa, pb, pad):
        def plane(p):
            p16 = p.reshape(NUM_SUBCORES, EDGES_PER_SUB)
            pad16 = jnp.broadcast_to(pad, (NUM_SUBCORES, PAD_E))
            return jnp.concatenate([p16, pad16], axis=1)
        return jnp.stack([plane(pa), plane(pb)]).reshape(-1, CHUNK)

    return planes(g0, g1, pad_g), planes(s0, s1, pad_s)


_BLK = 1000


def _dot16(a, b):
    return jnp.dot(a.astype(jnp.bfloat16), b.astype(jnp.bfloat16),
                   preferred_element_type=jnp.float32)


def _root1_body(x_ref, w_ref, b_ref, o_ref):
    o_ref[...] = _dot16(x_ref[...], w_ref[...]) + b_ref[...]


def _tc_root1(x, w_lin1, b1c):
    return pl.pallas_call(
        _root1_body,
        grid=(N // _BLK,),
        in_specs=[
            pl.BlockSpec((_BLK, D_IN), lambda i: (i, 0)),
            pl.BlockSpec((D_IN, D_HID), lambda i: (0, 0)),
            pl.BlockSpec((1, D_HID), lambda i: (0, 0)),
        ],
        out_specs=pl.BlockSpec((_BLK, D_HID), lambda i: (i, 0)),
        out_shape=jax.ShapeDtypeStruct((N, D_HID), jnp.float32),
    )(x, w_lin1, b1c)


def _mid_body(s_ref, r1_ref, wi1, wo1, wi2, wo2, wl2, b2, g_ref, r2_ref):
    h = jnp.maximum(
        0.5 * (_dot16(s_ref[0], wi1[...]) + _dot16(s_ref[1], wo1[...]))
        + r1_ref[...],
        0.0,
    )
    hb = h.astype(jnp.bfloat16)
    g_ref[0] = 0.5 * _dot16(hb, wi2[...])
    g_ref[1] = 0.5 * _dot16(hb, wo2[...])
    r2_ref[...] = _dot16(hb, wl2[...]) + b2[...]


def _tc_mid(s, r1, w_in1, w_out1, w_in2, w_out2, w_lin2, b2c):
    return pl.pallas_call(
        _mid_body,
        grid=(N // _BLK,),
        in_specs=[
            pl.BlockSpec((2, _BLK, D_IN), lambda i: (0, i, 0)),
            pl.BlockSpec((_BLK, D_HID), lambda i: (i, 0)),
            pl.BlockSpec((D_IN, D_HID), lambda i: (0, 0)),
            pl.BlockSpec((D_IN, D_HID), lambda i: (0, 0)),
            pl.BlockSpec((D_HID, D_OUT), lambda i: (0, 0)),
            pl.BlockSpec((D_HID, D_OUT), lambda i: (0, 0)),
            pl.BlockSpec((D_HID, D_OUT), lambda i: (0, 0)),
            pl.BlockSpec((1, D_OUT), lambda i: (0, 0)),
        ],
        out_specs=[
            pl.BlockSpec((2, _BLK, D_OUT), lambda i: (0, i, 0)),
            pl.BlockSpec((_BLK, D_OUT), lambda i: (i, 0)),
        ],
        out_shape=[
            jax.ShapeDtypeStruct((2, N, D_OUT), jnp.float32),
            jax.ShapeDtypeStruct((N, D_OUT), jnp.float32),
        ],
    )(s, r1, w_in1, w_out1, w_in2, w_out2, w_lin2, b2c)


def _final_body(t_ref, r2_ref, o_ref):
    o_ref[...] = t_ref[0] + t_ref[1] + r2_ref[...]


def _tc_final(t, r2):
    return pl.pallas_call(
        _final_body,
        grid=(N // _BLK,),
        in_specs=[
            pl.BlockSpec((2, _BLK, D_OUT), lambda i: (0, i, 0)),
            pl.BlockSpec((_BLK, D_OUT), lambda i: (i, 0)),
        ],
        out_specs=pl.BlockSpec((_BLK, D_OUT), lambda i: (i, 0)),
        out_shape=jax.ShapeDtypeStruct((N, D_OUT), jnp.float32),
    )(t, r2)


def kernel(x, edge_index, W_in1, b_in1, W_out1, b_out1, W_lin1, b_lin1,
           W_in2, b_in2, W_out2, b_out2, W_lin2, b_lin2):
    x = x.astype(jnp.float32)
    ei = edge_index.astype(jnp.int32)
    src, dst = ei[0], ei[1]
    gi1, si1 = _build_idx(src, dst, dst, src)
    gi2, si2 = _build_idx(src, dst, dst + N, src)
    zslab = jnp.zeros((ROWS_PER_SUB, 128), jnp.float32)
    b1c = (b_lin1 + 0.5 * (b_in1 + b_out1)).reshape(1, D_HID)
    b2c = (b_lin2 + 0.5 * (b_in2 + b_out2)).reshape(1, D_OUT)

    s_agg = _seg_sum_dual(x, gi1, si1, zslab)
    r1 = _tc_root1(x, W_lin1, b1c)
    g, r2 = _tc_mid(s_agg, r1, W_in1, W_out1, W_in2, W_out2, W_lin2, b2c)
    t_agg = _seg_sum_dual(g.reshape(2 * N, D_OUT), gi2, si2, zslab)
    return _tc_final(t_agg, r2)


# speedup vs baseline: 1.0034x; 1.0034x over previous
import functools

import jax
import jax.numpy as jnp
from jax import lax
from jax.experimental import pallas as pl
from jax.experimental.pallas import tpu as pltpu
from jax.experimental.pallas import tpu_sc as plsc

N = 10000
E = 320000
D_IN = 128
D_HID = 256
D_OUT = 128

NUM_SUBCORES = 16
EDGES_PER_SUB = E // NUM_SUBCORES
CHUNK = 128
IDX_BLOCK = 16
NUM_BLOCKS = 10
CHUNKS_PER_SUB = IDX_BLOCK * NUM_BLOCKS
EPS = CHUNK * CHUNKS_PER_SUB
PAD_E = EPS - EDGES_PER_SUB
NPAD = 10240
ROWS_PER_SUB = NPAD // NUM_SUBCORES


def _seg_sum_dual(data, gi, si, zslab, init0=None):
    mesh = plsc.VectorSubcoreMesh(core_axis_name="c", subcore_axis_name="s")
    with_init0 = init0 is not None
    if not with_init0:
        init0 = zslab

    @functools.partial(
        pl.kernel,
        mesh=mesh,
        out_type=jax.ShapeDtypeStruct((2, NPAD, 128), jnp.float32),
        scratch_types=[
            pltpu.VMEM((2, IDX_BLOCK, CHUNK), jnp.int32),
            pltpu.VMEM((2, IDX_BLOCK, CHUNK), jnp.int32),
            pltpu.VMEM((CHUNK, 128), jnp.float32),
            pltpu.VMEM((CHUNK, 128), jnp.float32),
            pltpu.VMEM_SHARED((NPAD, 128), jnp.float32),
            pltpu.SemaphoreType.DMA,
            pltpu.SemaphoreType.DMA,
            pltpu.SemaphoreType.DMA,
            pltpu.SemaphoreType.DMA,
        ],
    )
    def seg_kernel(data_hbm, gi_hbm, si_hbm, z_hbm, i0_hbm, out_hbm,
                   gi_v, si_v, rows0, rows1, acc_sh,
                   gsem0, gsem1, isem_g, isem_s):
        c = lax.axis_index("c")
        s = lax.axis_index("s")
        w = c * NUM_SUBCORES + s

        def i_start(b, p):
            row0 = w * CHUNKS_PER_SUB + b * IDX_BLOCK
            pltpu.async_copy(gi_hbm.at[pl.ds(row0, IDX_BLOCK)], gi_v.at[p], isem_g)
            pltpu.async_copy(si_hbm.at[pl.ds(row0, IDX_BLOCK)], si_v.at[p], isem_s)

        def i_wait(p):
            pltpu.make_async_copy(gi_hbm.at[pl.ds(0, IDX_BLOCK)], gi_v.at[p], isem_g).wait()
            pltpu.make_async_copy(si_hbm.at[pl.ds(0, IDX_BLOCK)], si_v.at[p], isem_s).wait()

        def g_start(idx_row, buf, sem):
            pltpu.async_copy(data_hbm.at[idx_row], buf, sem)

        def g_wait(buf, sem):
            pltpu.make_async_copy(data_hbm.at[gi_v.at[0, 0]], buf, sem).wait()

        i_start(0, 0)
        slab = acc_sh.at[pl.ds(s * ROWS_PER_SUB, ROWS_PER_SUB)]
        if with_init0:
            @pl.when(c == 0)
            def _():
                pltpu.sync_copy(
                    i0_hbm.at[pl.ds(s * ROWS_PER_SUB, ROWS_PER_SUB)], slab)

            @pl.when(c != 0)
            def _():
                pltpu.sync_copy(z_hbm, slab)
        else:
            pltpu.sync_copy(z_hbm, slab)
        plsc.subcore_barrier()
        i_wait(0)
        g_start(gi_v.at[0, 0], rows0, gsem0)
        g_start(gi_v.at[0, 1], rows1, gsem1)

        @pl.loop(0, NUM_BLOCKS)
        def _(b):
            p = lax.rem(b, 2)

            @pl.when(b + 1 < NUM_BLOCKS)
            def _():
                i_start(b + 1, 1 - p)

            @pl.loop(0, IDX_BLOCK, step=2)
            def _(k):
                bridge = jnp.logical_and(k + 2 >= IDX_BLOCK, b + 1 < NUM_BLOCKS)
                g_wait(rows0, gsem0)
                pltpu.sync_copy(rows0, acc_sh.at[si_v.at[p, k]], add=True)

                @pl.when(k + 2 < IDX_BLOCK)
                def _():
                    g_start(gi_v.at[p, k + 2], rows0, gsem0)

                @pl.when(bridge)
                def _():
                    i_wait(1 - p)
                    g_start(gi_v.at[1 - p, 0], rows0, gsem0)

                g_wait(rows1, gsem1)
                pltpu.sync_copy(rows1, acc_sh.at[si_v.at[p, k + 1]], add=True)

                @pl.when(k + 3 < IDX_BLOCK)
                def _():
                    g_start(gi_v.at[p, k + 3], rows1, gsem1)

                @pl.when(bridge)
                def _():
                    g_start(gi_v.at[1 - p, 1], rows1, gsem1)

        plsc.subcore_barrier()
        pltpu.sync_copy(
            acc_sh.at[pl.ds(s * ROWS_PER_SUB, ROWS_PER_SUB)],
            out_hbm.at[c, pl.ds(s * ROWS_PER_SUB, ROWS_PER_SUB)],
        )

    return seg_kernel(data, gi, si, zslab, init0)


def _build_idx(g0, s0, g1, s1):
    pad_g = (jnp.arange(PAD_E, dtype=jnp.int32) % 1024)
    pad_s = (N + jnp.arange(PAD_E, dtype=jnp.int32) % (NPAD - N))

    def planes(pa, pb, pad):
        def plane(p):
            p16 = p.reshape(NUM_SUBCORES, EDGES_PER_SUB)
            pad16 = jnp.broadcast_to(pad, (NUM_SUBCORES, PAD_E))
            return jnp.concatenate([p16, pad16], axis=1)
        return jnp.stack([plane(pa), plane(pb)]).reshape(-1, CHUNK)

    return planes(g0, g1, pad_g), planes(s0, s1, pad_s)


_BLK = 1000


def _dot16(a, b):
    return jnp.dot(a.astype(jnp.bfloat16), b.astype(jnp.bfloat16),
                   preferred_element_type=jnp.float32)


def _root1_body(x_ref, w_ref, b_ref, o_ref):
    o_ref[...] = _dot16(x_ref[...], w_ref[...]) + b_ref[...]


def _tc_root1(x, w_lin1, b1c):
    return pl.pallas_call(
        _root1_body,
        grid=(N // _BLK,),
        in_specs=[
            pl.BlockSpec((_BLK, D_IN), lambda i: (i, 0)),
            pl.BlockSpec((D_IN, D_HID), lambda i: (0, 0)),
            pl.BlockSpec((1, D_HID), lambda i: (0, 0)),
        ],
        out_specs=pl.BlockSpec((_BLK, D_HID), lambda i: (i, 0)),
        out_shape=jax.ShapeDtypeStruct((N, D_HID), jnp.float32),
    )(x, w_lin1, b1c)


def _mid_body(s_ref, r1_ref, wi1, wo1, wi2, wo2, wl2, b2, g_ref, r2_ref):
    h = jnp.maximum(
        0.5 * (_dot16(s_ref[0], wi1[...]) + _dot16(s_ref[1], wo1[...]))
        + r1_ref[...],
        0.0,
    )
    hb = h.astype(jnp.bfloat16)
    g_ref[0] = 0.5 * _dot16(hb, wi2[...])
    g_ref[1] = 0.5 * _dot16(hb, wo2[...])
    r2_ref[...] = _dot16(hb, wl2[...]) + b2[...]


def _tc_mid(s, r1, w_in1, w_out1, w_in2, w_out2, w_lin2, b2c):
    return pl.pallas_call(
        _mid_body,
        grid=(N // _BLK,),
        in_specs=[
            pl.BlockSpec((2, _BLK, D_IN), lambda i: (0, i, 0)),
            pl.BlockSpec((_BLK, D_HID), lambda i: (i, 0)),
            pl.BlockSpec((D_IN, D_HID), lambda i: (0, 0)),
            pl.BlockSpec((D_IN, D_HID), lambda i: (0, 0)),
            pl.BlockSpec((D_HID, D_OUT), lambda i: (0, 0)),
            pl.BlockSpec((D_HID, D_OUT), lambda i: (0, 0)),
            pl.BlockSpec((D_HID, D_OUT), lambda i: (0, 0)),
            pl.BlockSpec((1, D_OUT), lambda i: (0, 0)),
        ],
        out_specs=[
            pl.BlockSpec((2, _BLK, D_OUT), lambda i: (0, i, 0)),
            pl.BlockSpec((_BLK, D_OUT), lambda i: (i, 0)),
        ],
        out_shape=[
            jax.ShapeDtypeStruct((2, N, D_OUT), jnp.float32),
            jax.ShapeDtypeStruct((NPAD, D_OUT), jnp.float32),
        ],
    )(s, r1, w_in1, w_out1, w_in2, w_out2, w_lin2, b2c)


def _final_body(t_ref, o_ref):
    o_ref[...] = t_ref[0] + t_ref[1]


def _tc_final(t):
    return pl.pallas_call(
        _final_body,
        grid=(N // _BLK,),
        in_specs=[pl.BlockSpec((2, _BLK, D_OUT), lambda i: (0, i, 0))],
        out_specs=pl.BlockSpec((_BLK, D_OUT), lambda i: (i, 0)),
        out_shape=jax.ShapeDtypeStruct((N, D_OUT), jnp.float32),
    )(t)


def kernel(x, edge_index, W_in1, b_in1, W_out1, b_out1, W_lin1, b_lin1,
           W_in2, b_in2, W_out2, b_out2, W_lin2, b_lin2):
    x = x.astype(jnp.float32)
    ei = edge_index.astype(jnp.int32)
    src, dst = ei[0], ei[1]
    gi1, si1 = _build_idx(src, dst, dst, src)
    gi2, si2 = _build_idx(src, dst, dst + N, src)
    zslab = jnp.zeros((ROWS_PER_SUB, 128), jnp.float32)
    b1c = (b_lin1 + 0.5 * (b_in1 + b_out1)).reshape(1, D_HID)
    b2c = (b_lin2 + 0.5 * (b_in2 + b_out2)).reshape(1, D_OUT)

    s_agg = _seg_sum_dual(x, gi1, si1, zslab)
    r1 = _tc_root1(x, W_lin1, b1c)
    g, r2 = _tc_mid(s_agg, r1, W_in1, W_out1, W_in2, W_out2, W_lin2, b2c)
    t_agg = _seg_sum_dual(g.reshape(2 * N, D_OUT), gi2, si2, zslab, init0=r2)
    return _tc_final(t_agg)


# speedup vs baseline: 1.0153x; 1.0119x over previous
import functools

import jax
import jax.numpy as jnp
from jax import lax
from jax.experimental import pallas as pl
from jax.experimental.pallas import tpu as pltpu
from jax.experimental.pallas import tpu_sc as plsc

N = 10000
E = 320000
D_IN = 128
D_HID = 256
D_OUT = 128

NUM_SUBCORES = 16
EDGES_PER_SUB = E // NUM_SUBCORES
CHUNK = 128
IDX_BLOCK = 16
NUM_BLOCKS = 10
CHUNKS_PER_SUB = IDX_BLOCK * NUM_BLOCKS
EPS = CHUNK * CHUNKS_PER_SUB
PAD_E = EPS - EDGES_PER_SUB
NPAD = 10240
ROWS_PER_SUB = NPAD // NUM_SUBCORES


def _seg_sum_dual(data, gi, si, zslab, init0=None):
    mesh = plsc.VectorSubcoreMesh(core_axis_name="c", subcore_axis_name="s")
    with_init0 = init0 is not None
    if not with_init0:
        init0 = zslab

    @functools.partial(
        pl.kernel,
        mesh=mesh,
        out_type=jax.ShapeDtypeStruct((2, NPAD, 128), jnp.float32),
        scratch_types=[
            pltpu.VMEM((2, IDX_BLOCK, CHUNK), jnp.int32),
            pltpu.VMEM((2, IDX_BLOCK, CHUNK), jnp.int32),
            pltpu.VMEM((CHUNK, 128), jnp.float32),
            pltpu.VMEM((CHUNK, 128), jnp.float32),
            pltpu.VMEM_SHARED((NPAD, 128), jnp.float32),
            pltpu.SemaphoreType.DMA,
            pltpu.SemaphoreType.DMA,
            pltpu.SemaphoreType.DMA,
            pltpu.SemaphoreType.DMA,
        ],
    )
    def seg_kernel(data_hbm, gi_hbm, si_hbm, z_hbm, i0_hbm, out_hbm,
                   gi_v, si_v, rows0, rows1, acc_sh,
                   gsem0, gsem1, isem_g, isem_s):
        c = lax.axis_index("c")
        s = lax.axis_index("s")
        w = c * NUM_SUBCORES + s

        def i_start(b, p):
            row0 = w * CHUNKS_PER_SUB + b * IDX_BLOCK
            pltpu.async_copy(gi_hbm.at[pl.ds(row0, IDX_BLOCK)], gi_v.at[p], isem_g)
            pltpu.async_copy(si_hbm.at[pl.ds(row0, IDX_BLOCK)], si_v.at[p], isem_s)

        def i_wait(p):
            pltpu.make_async_copy(gi_hbm.at[pl.ds(0, IDX_BLOCK)], gi_v.at[p], isem_g).wait()
            pltpu.make_async_copy(si_hbm.at[pl.ds(0, IDX_BLOCK)], si_v.at[p], isem_s).wait()

        def g_start(idx_row, buf, sem):
            pltpu.async_copy(data_hbm.at[idx_row], buf, sem)

        def g_wait(buf, sem):
            pltpu.make_async_copy(data_hbm.at[gi_v.at[0, 0]], buf, sem).wait()

        i_start(0, 0)
        slab = acc_sh.at[pl.ds(s * ROWS_PER_SUB, ROWS_PER_SUB)]
        if with_init0:
            @pl.when(c == 0)
            def _():
                pltpu.sync_copy(
                    i0_hbm.at[pl.ds(s * ROWS_PER_SUB, ROWS_PER_SUB)], slab)

            @pl.when(c != 0)
            def _():
                pltpu.sync_copy(z_hbm, slab)
        else:
            pltpu.sync_copy(z_hbm, slab)
        plsc.subcore_barrier()
        i_wait(0)
        g_start(gi_v.at[0, 0], rows0, gsem0)
        g_start(gi_v.at[0, 1], rows1, gsem1)

        @pl.loop(0, NUM_BLOCKS)
        def _(b):
            p = lax.rem(b, 2)

            @pl.when(b + 1 < NUM_BLOCKS)
            def _():
                i_start(b + 1, 1 - p)

            @pl.loop(0, IDX_BLOCK, step=2)
            def _(k):
                bridge = jnp.logical_and(k + 2 >= IDX_BLOCK, b + 1 < NUM_BLOCKS)
                g_wait(rows0, gsem0)
                pltpu.sync_copy(rows0, acc_sh.at[si_v.at[p, k]], add=True)

                @pl.when(k + 2 < IDX_BLOCK)
                def _():
                    g_start(gi_v.at[p, k + 2], rows0, gsem0)

                @pl.when(bridge)
                def _():
                    i_wait(1 - p)
                    g_start(gi_v.at[1 - p, 0], rows0, gsem0)

                g_wait(rows1, gsem1)
                pltpu.sync_copy(rows1, acc_sh.at[si_v.at[p, k + 1]], add=True)

                @pl.when(k + 3 < IDX_BLOCK)
                def _():
                    g_start(gi_v.at[p, k + 3], rows1, gsem1)

                @pl.when(bridge)
                def _():
                    g_start(gi_v.at[1 - p, 1], rows1, gsem1)

        plsc.subcore_barrier()
        pltpu.sync_copy(
            acc_sh.at[pl.ds(s * ROWS_PER_SUB, ROWS_PER_SUB)],
            out_hbm.at[c, pl.ds(s * ROWS_PER_SUB, ROWS_PER_SUB)],
        )

    return seg_kernel(data, gi, si, zslab, init0)


def _build_idx(g0, s0, g1, s1):
    pad_g = (jnp.arange(PAD_E, dtype=jnp.int32) % 1024)
    pad_s = (N + jnp.arange(PAD_E, dtype=jnp.int32) % (NPAD - N))

    def planes(pa, pb, pad):
        def plane(p):
            p16 = p.reshape(NUM_SUBCORES, EDGES_PER_SUB)
            pad16 = jnp.broadcast_to(pad, (NUM_SUBCORES, PAD_E))
            return jnp.concatenate([p16, pad16], axis=1)
        return jnp.stack([plane(pa), plane(pb)]).reshape(-1, CHUNK)

    return planes(g0, g1, pad_g), planes(s0, s1, pad_s)


_BLK = 2000


def _dot16(a, b):
    return jnp.dot(a.astype(jnp.bfloat16), b.astype(jnp.bfloat16),
                   preferred_element_type=jnp.float32)


def _root1_body(x_ref, w_ref, b_ref, o_ref):
    o_ref[...] = _dot16(x_ref[...], w_ref[...]) + b_ref[...]


def _tc_root1(x, w_lin1, b1c):
    return pl.pallas_call(
        _root1_body,
        grid=(N // _BLK,),
        in_specs=[
            pl.BlockSpec((_BLK, D_IN), lambda i: (i, 0)),
            pl.BlockSpec((D_IN, D_HID), lambda i: (0, 0)),
            pl.BlockSpec((1, D_HID), lambda i: (0, 0)),
        ],
        out_specs=pl.BlockSpec((_BLK, D_HID), lambda i: (i, 0)),
        out_shape=jax.ShapeDtypeStruct((N, D_HID), jnp.float32),
    )(x, w_lin1, b1c)


def _mid_body(s_ref, r1_ref, wi1, wo1, wi2, wo2, wl2, b2, g_ref, r2_ref):
    h = jnp.maximum(
        0.5 * (_dot16(s_ref[0], wi1[...]) + _dot16(s_ref[1], wo1[...]))
        + r1_ref[...],
        0.0,
    )
    hb = h.astype(jnp.bfloat16)
    g_ref[0] = 0.5 * _dot16(hb, wi2[...])
    g_ref[1] = 0.5 * _dot16(hb, wo2[...])
    r2_ref[...] = _dot16(hb, wl2[...]) + b2[...]


def _tc_mid(s, r1, w_in1, w_out1, w_in2, w_out2, w_lin2, b2c):
    return pl.pallas_call(
        _mid_body,
        grid=(N // _BLK,),
        in_specs=[
            pl.BlockSpec((2, _BLK, D_IN), lambda i: (0, i, 0)),
            pl.BlockSpec((_BLK, D_HID), lambda i: (i, 0)),
            pl.BlockSpec((D_IN, D_HID), lambda i: (0, 0)),
            pl.BlockSpec((D_IN, D_HID), lambda i: (0, 0)),
            pl.BlockSpec((D_HID, D_OUT), lambda i: (0, 0)),
            pl.BlockSpec((D_HID, D_OUT), lambda i: (0, 0)),
            pl.BlockSpec((D_HID, D_OUT), lambda i: (0, 0)),
            pl.BlockSpec((1, D_OUT), lambda i: (0, 0)),
        ],
        out_specs=[
            pl.BlockSpec((2, _BLK, D_OUT), lambda i: (0, i, 0)),
            pl.BlockSpec((_BLK, D_OUT), lambda i: (i, 0)),
        ],
        out_shape=[
            jax.ShapeDtypeStruct((2, N, D_OUT), jnp.float32),
            jax.ShapeDtypeStruct((NPAD, D_OUT), jnp.float32),
        ],
    )(s, r1, w_in1, w_out1, w_in2, w_out2, w_lin2, b2c)


def _final_body(t_ref, o_ref):
    o_ref[...] = t_ref[0] + t_ref[1]


def _tc_final(t):
    return pl.pallas_call(
        _final_body,
        grid=(N // _BLK,),
        in_specs=[pl.BlockSpec((2, _BLK, D_OUT), lambda i: (0, i, 0))],
        out_specs=pl.BlockSpec((_BLK, D_OUT), lambda i: (i, 0)),
        out_shape=jax.ShapeDtypeStruct((N, D_OUT), jnp.float32),
    )(t)


def kernel(x, edge_index, W_in1, b_in1, W_out1, b_out1, W_lin1, b_lin1,
           W_in2, b_in2, W_out2, b_out2, W_lin2, b_lin2):
    x = x.astype(jnp.float32)
    ei = edge_index.astype(jnp.int32)
    src, dst = ei[0], ei[1]
    gi1, si1 = _build_idx(src, dst, dst, src)
    gi2, si2 = _build_idx(src, dst, dst + N, src)
    zslab = jnp.zeros((ROWS_PER_SUB, 128), jnp.float32)
    b1c = (b_lin1 + 0.5 * (b_in1 + b_out1)).reshape(1, D_HID)
    b2c = (b_lin2 + 0.5 * (b_in2 + b_out2)).reshape(1, D_OUT)

    s_agg = _seg_sum_dual(x, gi1, si1, zslab)
    r1 = _tc_root1(x, W_lin1, b1c)
    g, r2 = _tc_mid(s_agg, r1, W_in1, W_out1, W_in2, W_out2, W_lin2, b2c)
    t_agg = _seg_sum_dual(g.reshape(2 * N, D_OUT), gi2, si2, zslab, init0=r2)
    return _tc_final(t_agg)


# speedup vs baseline: 1.0242x; 1.0088x over previous
import functools

import jax
import jax.numpy as jnp
from jax import lax
from jax.experimental import pallas as pl
from jax.experimental.pallas import tpu as pltpu
from jax.experimental.pallas import tpu_sc as plsc

N = 10000
E = 320000
D_IN = 128
D_HID = 256
D_OUT = 128

NUM_SUBCORES = 16
EDGES_PER_SUB = E // NUM_SUBCORES
CHUNK = 128
IDX_BLOCK = 16
NUM_BLOCKS = 10
CHUNKS_PER_SUB = IDX_BLOCK * NUM_BLOCKS
EPS = CHUNK * CHUNKS_PER_SUB
PAD_E = EPS - EDGES_PER_SUB
NPAD = 10240
ROWS_PER_SUB = NPAD // NUM_SUBCORES


def _seg_sum_dual(data, gi, si, zslab, init0=None):
    mesh = plsc.VectorSubcoreMesh(core_axis_name="c", subcore_axis_name="s")
    with_init0 = init0 is not None
    if not with_init0:
        init0 = zslab

    @functools.partial(
        pl.kernel,
        mesh=mesh,
        out_type=jax.ShapeDtypeStruct((2, NPAD, 128), jnp.float32),
        scratch_types=[
            pltpu.VMEM((2, IDX_BLOCK, CHUNK), jnp.int32),
            pltpu.VMEM((2, IDX_BLOCK, CHUNK), jnp.int32),
            pltpu.VMEM((CHUNK, 128), jnp.float32),
            pltpu.VMEM((CHUNK, 128), jnp.float32),
            pltpu.VMEM_SHARED((NPAD, 128), jnp.float32),
            pltpu.SemaphoreType.DMA,
            pltpu.SemaphoreType.DMA,
            pltpu.SemaphoreType.DMA,
            pltpu.SemaphoreType.DMA,
        ],
    )
    def seg_kernel(data_hbm, gi_hbm, si_hbm, z_hbm, i0_hbm, out_hbm,
                   gi_v, si_v, rows0, rows1, acc_sh,
                   gsem0, gsem1, isem_g, isem_s):
        c = lax.axis_index("c")
        s = lax.axis_index("s")
        w = c * NUM_SUBCORES + s

        def i_start(b, p):
            row0 = w * CHUNKS_PER_SUB + b * IDX_BLOCK
            pltpu.async_copy(gi_hbm.at[pl.ds(row0, IDX_BLOCK)], gi_v.at[p], isem_g)
            pltpu.async_copy(si_hbm.at[pl.ds(row0, IDX_BLOCK)], si_v.at[p], isem_s)

        def i_wait(p):
            pltpu.make_async_copy(gi_hbm.at[pl.ds(0, IDX_BLOCK)], gi_v.at[p], isem_g).wait()
            pltpu.make_async_copy(si_hbm.at[pl.ds(0, IDX_BLOCK)], si_v.at[p], isem_s).wait()

        def g_start(idx_row, buf, sem):
            pltpu.async_copy(data_hbm.at[idx_row], buf, sem)

        def g_wait(buf, sem):
            pltpu.make_async_copy(data_hbm.at[gi_v.at[0, 0]], buf, sem).wait()

        i_start(0, 0)
        slab = acc_sh.at[pl.ds(s * ROWS_PER_SUB, ROWS_PER_SUB)]
        if with_init0:
            @pl.when(c == 0)
            def _():
                pltpu.sync_copy(
                    i0_hbm.at[pl.ds(s * ROWS_PER_SUB, ROWS_PER_SUB)], slab)

            @pl.when(c != 0)
            def _():
                pltpu.sync_copy(z_hbm, slab)
        else:
            pltpu.sync_copy(z_hbm, slab)
        plsc.subcore_barrier()
        i_wait(0)
        g_start(gi_v.at[0, 0], rows0, gsem0)
        g_start(gi_v.at[0, 1], rows1, gsem1)

        @pl.loop(0, NUM_BLOCKS)
        def _(b):
            p = lax.rem(b, 2)

            @pl.when(b + 1 < NUM_BLOCKS)
            def _():
                i_start(b + 1, 1 - p)

            @pl.loop(0, IDX_BLOCK, step=2)
            def _(k):
                bridge = jnp.logical_and(k + 2 >= IDX_BLOCK, b + 1 < NUM_BLOCKS)
                g_wait(rows0, gsem0)
                pltpu.sync_copy(rows0, acc_sh.at[si_v.at[p, k]], add=True)

                @pl.when(k + 2 < IDX_BLOCK)
                def _():
                    g_start(gi_v.at[p, k + 2], rows0, gsem0)

                @pl.when(bridge)
                def _():
                    i_wait(1 - p)
                    g_start(gi_v.at[1 - p, 0], rows0, gsem0)

                g_wait(rows1, gsem1)
                pltpu.sync_copy(rows1, acc_sh.at[si_v.at[p, k + 1]], add=True)

                @pl.when(k + 3 < IDX_BLOCK)
                def _():
                    g_start(gi_v.at[p, k + 3], rows1, gsem1)

                @pl.when(bridge)
                def _():
                    g_start(gi_v.at[1 - p, 1], rows1, gsem1)

        plsc.subcore_barrier()
        pltpu.sync_copy(
            acc_sh.at[pl.ds(s * ROWS_PER_SUB, ROWS_PER_SUB)],
            out_hbm.at[c, pl.ds(s * ROWS_PER_SUB, ROWS_PER_SUB)],
        )

    return seg_kernel(data, gi, si, zslab, init0)


def _build_idx(g0, s0, g1, s1):
    pad_g = (jnp.arange(PAD_E, dtype=jnp.int32) % 1024)
    pad_s = (N + jnp.arange(PAD_E, dtype=jnp.int32) % (NPAD - N))

    def planes(pa, pb, pad):
        def plane(p):
            p16 = p.reshape(NUM_SUBCORES, EDGES_PER_SUB)
            pad16 = jnp.broadcast_to(pad, (NUM_SUBCORES, PAD_E))
            return jnp.concatenate([p16, pad16], axis=1)
        return jnp.stack([plane(pa), plane(pb)]).reshape(-1, CHUNK)

    return planes(g0, g1, pad_g), planes(s0, s1, pad_s)


_BLK = 5000


def _dot16(a, b):
    return jnp.dot(a.astype(jnp.bfloat16), b.astype(jnp.bfloat16),
                   preferred_element_type=jnp.float32)


def _root1_body(x_ref, w_ref, b_ref, o_ref):
    o_ref[...] = _dot16(x_ref[...], w_ref[...]) + b_ref[...]


def _tc_root1(x, w_lin1, b1c):
    return pl.pallas_call(
        _root1_body,
        grid=(N // _BLK,),
        in_specs=[
            pl.BlockSpec((_BLK, D_IN), lambda i: (i, 0)),
            pl.BlockSpec((D_IN, D_HID), lambda i: (0, 0)),
            pl.BlockSpec((1, D_HID), lambda i: (0, 0)),
        ],
        out_specs=pl.BlockSpec((_BLK, D_HID), lambda i: (i, 0)),
        out_shape=jax.ShapeDtypeStruct((N, D_HID), jnp.float32),
    )(x, w_lin1, b1c)


def _mid_body(s_ref, r1_ref, wi1, wo1, wi2, wo2, wl2, b2, g_ref, r2_ref):
    h = jnp.maximum(
        0.5 * (_dot16(s_ref[0], wi1[...]) + _dot16(s_ref[1], wo1[...]))
        + r1_ref[...],
        0.0,
    )
    hb = h.astype(jnp.bfloat16)
    g_ref[0] = 0.5 * _dot16(hb, wi2[...])
    g_ref[1] = 0.5 * _dot16(hb, wo2[...])
    r2_ref[...] = _dot16(hb, wl2[...]) + b2[...]


def _tc_mid(s, r1, w_in1, w_out1, w_in2, w_out2, w_lin2, b2c):
    return pl.pallas_call(
        _mid_body,
        grid=(N // _BLK,),
        in_specs=[
            pl.BlockSpec((2, _BLK, D_IN), lambda i: (0, i, 0)),
            pl.BlockSpec((_BLK, D_HID), lambda i: (i, 0)),
            pl.BlockSpec((D_IN, D_HID), lambda i: (0, 0)),
            pl.BlockSpec((D_IN, D_HID), lambda i: (0, 0)),
            pl.BlockSpec((D_HID, D_OUT), lambda i: (0, 0)),
            pl.BlockSpec((D_HID, D_OUT), lambda i: (0, 0)),
            pl.BlockSpec((D_HID, D_OUT), lambda i: (0, 0)),
            pl.BlockSpec((1, D_OUT), lambda i: (0, 0)),
        ],
        out_specs=[
            pl.BlockSpec((2, _BLK, D_OUT), lambda i: (0, i, 0)),
            pl.BlockSpec((_BLK, D_OUT), lambda i: (i, 0)),
        ],
        out_shape=[
            jax.ShapeDtypeStruct((2, N, D_OUT), jnp.float32),
            jax.ShapeDtypeStruct((NPAD, D_OUT), jnp.float32),
        ],
    )(s, r1, w_in1, w_out1, w_in2, w_out2, w_lin2, b2c)


def _final_body(t_ref, o_ref):
    o_ref[...] = t_ref[0] + t_ref[1]


def _tc_final(t):
    return pl.pallas_call(
        _final_body,
        grid=(N // _BLK,),
        in_specs=[pl.BlockSpec((2, _BLK, D_OUT), lambda i: (0, i, 0))],
        out_specs=pl.BlockSpec((_BLK, D_OUT), lambda i: (i, 0)),
        out_shape=jax.ShapeDtypeStruct((N, D_OUT), jnp.float32),
    )(t)


def kernel(x, edge_index, W_in1, b_in1, W_out1, b_out1, W_lin1, b_lin1,
           W_in2, b_in2, W_out2, b_out2, W_lin2, b_lin2):
    x = x.astype(jnp.float32)
    ei = edge_index.astype(jnp.int32)
    src, dst = ei[0], ei[1]
    gi1, si1 = _build_idx(src, dst, dst, src)
    gi2, si2 = _build_idx(src, dst, dst + N, src)
    zslab = jnp.zeros((ROWS_PER_SUB, 128), jnp.float32)
    b1c = (b_lin1 + 0.5 * (b_in1 + b_out1)).reshape(1, D_HID)
    b2c = (b_lin2 + 0.5 * (b_in2 + b_out2)).reshape(1, D_OUT)

    s_agg = _seg_sum_dual(x, gi1, si1, zslab)
    r1 = _tc_root1(x, W_lin1, b1c)
    g, r2 = _tc_mid(s_agg, r1, W_in1, W_out1, W_in2, W_out2, W_lin2, b2c)
    t_agg = _seg_sum_dual(g.reshape(2 * N, D_OUT), gi2, si2, zslab, init0=r2)
    return _tc_final(t_agg)
